# baseline (speedup 1.0000x reference)
import jax
import jax.numpy as jnp
from jax.experimental import pallas as pl
from jax.experimental.pallas import tpu as pltpu

_BG = 8


def _concat_kernel(sid_ref, emg_ref, table_ref, out_ref):
    F = emg_ref.shape[-1]
    T = emg_ref.shape[1]
    E = table_ref.shape[-1]
    out_ref[:, :, :F] = emg_ref[...]
    base = pl.program_id(0) * _BG
    for j in range(_BG):
        sid = sid_ref[base + j]
        row = table_ref[pl.ds(sid, 1), :]
        out_ref[j, :, F:] = jnp.broadcast_to(row, (T, E))


def kernel(emg_features, session_ids, table):
    B, T, F = emg_features.shape
    N, E = table.shape
    sids = session_ids.astype(jnp.int32)
    return pl.pallas_call(
        _concat_kernel,
        grid_spec=pltpu.PrefetchScalarGridSpec(
            num_scalar_prefetch=1,
            grid=(B // _BG,),
            in_specs=[
                pl.BlockSpec((_BG, T, F), lambda i, sid: (i, 0, 0)),
                pl.BlockSpec((N, E), lambda i, sid: (0, 0)),
            ],
            out_specs=pl.BlockSpec((_BG, T, F + E), lambda i, sid: (i, 0, 0)),
        ),
        out_shape=jax.ShapeDtypeStruct((B, T, F + E), jnp.float32),
    )(sids, emg_features, table)


# speedup vs baseline: 1.1106x; 1.1106x over previous
import jax
import jax.numpy as jnp
from jax.experimental import pallas as pl
from jax.experimental.pallas import tpu as pltpu

_BG = 32


def _concat_kernel(sid_ref, emg_ref, table_ref, out_ref):
    F = emg_ref.shape[-1]
    T = emg_ref.shape[1]
    E = table_ref.shape[-1]
    out_ref[:, :, :F] = emg_ref[...]
    base = pl.program_id(0) * _BG
    for j in range(_BG):
        sid = sid_ref[base + j]
        row = table_ref[pl.ds(sid, 1), :]
        out_ref[j, :, F:] = jnp.broadcast_to(row, (T, E))


def kernel(emg_features, session_ids, table):
    B, T, F = emg_features.shape
    N, E = table.shape
    sids = session_ids.astype(jnp.int32)
    return pl.pallas_call(
        _concat_kernel,
        grid_spec=pltpu.PrefetchScalarGridSpec(
            num_scalar_prefetch=1,
            grid=(B // _BG,),
            in_specs=[
                pl.BlockSpec((_BG, T, F), lambda i, sid: (i, 0, 0)),
                pl.BlockSpec((N, E), lambda i, sid: (0, 0)),
            ],
            out_specs=pl.BlockSpec((_BG, T, F + E), lambda i, sid: (i, 0, 0)),
        ),
        out_shape=jax.ShapeDtypeStruct((B, T, F + E), jnp.float32),
    )(sids, emg_features, table)
